# baseline (speedup 1.0000x reference)
import jax
import jax.numpy as jnp
from jax.experimental import pallas as pl
from jax.experimental.pallas import tpu as pltpu


def _pos_kernel(col_hbm, row_hbm, out_ref, col_v, row_v, scr, sem, insem):
    w, d = col_v.shape
    h, _ = row_v.shape
    b = out_ref.shape[0]
    in0 = pltpu.make_async_copy(col_hbm.at[pl.ds(0, w)], col_v, insem.at[0])
    in1 = pltpu.make_async_copy(row_hbm.at[pl.ds(0, h)], row_v, insem.at[1])
    in0.start()
    in1.start()
    in0.wait()
    in1.wait()
    scr[:, :, 0:d] = jnp.broadcast_to(col_v[...][None, :, :], (h, w, d))
    scr[:, :, d:2 * d] = jnp.broadcast_to(row_v[...][:, None, :], (h, w, d))
    copies = [
        pltpu.make_async_copy(scr, out_ref.at[i], sem.at[i]) for i in range(b)
    ]
    for cp in copies:
        cp.start()
    for cp in copies:
        cp.wait()


def kernel(tensor_list, row_embed, col_embed):
    b = tensor_list.shape[0]
    h, w = tensor_list.shape[-2], tensor_list.shape[-1]
    d = col_embed.shape[-1]
    out = pl.pallas_call(
        _pos_kernel,
        out_shape=jax.ShapeDtypeStruct((b, h, w, 2 * d), jnp.float32),
        in_specs=[
            pl.BlockSpec(memory_space=pl.ANY),
            pl.BlockSpec(memory_space=pl.ANY),
        ],
        out_specs=pl.BlockSpec(memory_space=pl.ANY),
        scratch_shapes=[
            pltpu.VMEM((w, d), jnp.float32),
            pltpu.VMEM((h, d), jnp.float32),
            pltpu.VMEM((h, w, 2 * d), jnp.float32),
            pltpu.SemaphoreType.DMA((b,)),
            pltpu.SemaphoreType.DMA((2,)),
        ],
    )(col_embed, row_embed)
    return jnp.transpose(out, (0, 3, 1, 2))
